# baseline (speedup 1.0000x reference)
import functools

import jax
import jax.numpy as jnp
from jax import lax
from jax.experimental import pallas as pl
from jax.experimental.pallas import tpu as pltpu
from jax.experimental.pallas import tpu_sc as plsc

_B = 512
_NG = 3008
_DD = 2048
_K = 6
_T1, _T2 = 1000, 250
_F1, _F2 = 16, 8
_T1P = 1024
_T2P = 256
_NW = 32
_NV = _B // 16


def _splat(ref, idx):
  return plsc.load_gather(ref, [jnp.full((16,), idx, jnp.int32)])


def _tanh16(v):
  a = jnp.abs(v + v)
  e = jnp.exp(a)
  r = 2.0 / (e + 1.0)
  return jnp.where(v < 0.0, r - 1.0, 1.0 - r)


def _rsqrt16(x):
  i = plsc.bitcast(x, jnp.int32)
  y = plsc.bitcast(jnp.int32(0x5F3759DF) - (i >> 1), jnp.float32)
  for _ in range(4):
    y = y * (1.5 - 0.5 * x * y * y)
  return y


def _sc_layer(table, idx, par, n_in, terms, tpc, par_w):
  tpw = terms // _NW
  cpw = tpw // tpc
  npairs = cpw // 2
  rows_n = tpc * n_in
  wb = 24 // (tpc * _K)
  wpairs = wb // 2 if wb >= 2 else 1
  mesh = plsc.VectorSubcoreMesh(core_axis_name="c", subcore_axis_name="s")

  @functools.partial(
      pl.kernel,
      out_type=jax.ShapeDtypeStruct((terms * _K, _B), jnp.float32),
      mesh=mesh,
      compiler_params=pltpu.CompilerParams(needs_layout_passes=False,
                                           disable_bounds_checks=True),
      scratch_types=[
          pltpu.VMEM((cpw, rows_n), jnp.int32),
          pltpu.VMEM((tpw * par_w,), jnp.float32),
          pltpu.VMEM((rows_n, _B), jnp.float32),
          pltpu.VMEM((rows_n, _B), jnp.float32),
          pltpu.VMEM((24, _B), jnp.float32),
          pltpu.SemaphoreType.DMA,
          pltpu.SemaphoreType.DMA,
      ],
  )
  def layer_k(table_hbm, idx_hbm, par_hbm, out_hbm,
              idx_v, par_v, buf0, buf1, zbuf, sem0, sem1):
    wid = lax.axis_index("s") * 2 + lax.axis_index("c")
    pltpu.sync_copy(idx_hbm.at[pl.ds(wid * cpw, cpw)], idx_v)
    pltpu.sync_copy(par_hbm.at[pl.ds(wid * tpw * par_w, tpw * par_w)], par_v)
    bufs = (buf0, buf1)
    sems = (sem0, sem1)
    pltpu.async_copy(table_hbm.at[idx_v.at[0]], buf0, sem0)
    pltpu.async_copy(table_hbm.at[idx_v.at[1]], buf1, sem1)

    u = 8 if n_in % 8 == 0 else 6

    def process_term(rows_ref, prow, tl, zbase):
      pbase = prow * par_w
      ivec = jnp.full((16,), pbase, jnp.int32)

      def contract_cb(cb, carry):
        bias = [plsc.load_gather(par_v, [ivec + (n_in * _K + k)])
                for k in range(_K)]
        acc0 = tuple(bias[k] for k in range(_K) for _ in range(8))

        def jqb_body(jqi, accs):
          accs = list(accs)
          iv2 = ivec + jqi * (6 * u)
          rbase = tl * n_in + jqi * u
          for jj in range(u):
            w = [plsc.load_gather(par_v, [iv2 + (jj * 6 + k)])
                 for k in range(_K)]
            for c in range(8):
              r = rows_ref[rbase + jj, pl.ds(cb * 128 + c * 16, 16)]
              for k in range(_K):
                accs[k * 8 + c] = accs[k * 8 + c] + w[k] * r
          return tuple(accs)

        accs = lax.fori_loop(0, n_in // u, jqb_body, acc0)
        for k in range(_K):
          for c in range(8):
            zbuf[zbase + tl * _K + k, pl.ds(cb * 128 + c * 16, 16)] = accs[k * 8 + c]
        return carry

      def norm_row(r, carry):
        row = zbase + tl * _K + r
        s1 = jnp.zeros((16,), jnp.float32)
        s2 = jnp.zeros((16,), jnp.float32)
        for c in range(_NV):
          t = _tanh16(zbuf[row, pl.ds(c * 16, 16)])
          zbuf[row, pl.ds(c * 16, 16)] = t
          s1 = s1 + t
          s2 = s2 + t * t
        m = jnp.full((16,), jnp.sum(s1) * (1.0 / _B), jnp.float32)
        v = jnp.full((16,), jnp.sum(s2) * (1.0 / _B), jnp.float32) - m * m
        inv = _rsqrt16(v + 1e-5)
        ga = _splat(par_v, prow * par_w + n_in * _K + _K + r)
        be = _splat(par_v, prow * par_w + n_in * _K + 2 * _K + r)
        aa = ga * inv
        cc = be - m * aa
        for c in range(_NV):
          zbuf[row, pl.ds(c * 16, 16)] = aa * zbuf[row, pl.ds(c * 16, 16)] + cc
        return carry

      lax.fori_loop(0, 4, contract_cb, 0)
      lax.fori_loop(0, _K, norm_row, 0)

    def pair_body(p, carry):
      for half in range(2):
        ch = 2 * p + half
        zbase = (ch % wb) * tpc * _K
        pltpu.make_async_copy(
            table_hbm.at[pl.ds(0, rows_n)], bufs[half], sems[half]).wait()
        for tl in range(tpc):
          process_term(bufs[half], ch * tpc + tl, tl, zbase)

        @pl.when(p < npairs - 1)
        def _():
          nxt = ch + 2
          pltpu.async_copy(table_hbm.at[idx_v.at[nxt]], bufs[half], sems[half])

        if half == 1:
          @pl.when(p % wpairs == wpairs - 1)
          def _():
            base_chunk = ch + 1 - wb
            off = pl.multiple_of((wid * cpw + base_chunk) * tpc * _K, 8)
            pltpu.sync_copy(zbuf, out_hbm.at[pl.ds(off, 24)])
      return carry

    lax.fori_loop(0, npairs, pair_body, 0)

  return layer_k(table, idx, par)


def _bn_lanes(z, ga, be):
  m = jnp.mean(z, axis=-1, keepdims=True)
  dd = z - m
  v = jnp.mean(dd * dd, axis=-1, keepdims=True)
  return ga * dd * lax.rsqrt(v + 1e-5) + be


def _head_body(h2_ref, dr_ref, w3_ref, b3_ref, g3_ref, e3_ref,
               wd1_ref, bd1_ref, gd1_ref, ed1_ref,
               wd2_ref, bd2_ref, gd2_ref, ed2_ref,
               wd3_ref, bd3_ref, gd3_ref, ed3_ref,
               wf_ref, bf_ref, gf_ref, ef_ref,
               wa_ref, ba_ref, wo_ref, bo_ref, o_ref):
  def dot(a, b):
    return lax.dot(a, b, precision=lax.Precision.HIGHEST,
                   preferred_element_type=jnp.float32)

  h3 = _bn_lanes(jnp.tanh(dot(w3_ref[...], h2_ref[...]) + b3_ref[...]),
                 g3_ref[...], e3_ref[...])
  d = dr_ref[...]
  d = _bn_lanes(jnp.tanh(dot(wd1_ref[...], d) + bd1_ref[...]),
                gd1_ref[...], ed1_ref[...])
  d = _bn_lanes(jnp.tanh(dot(wd2_ref[...], d) + bd2_ref[...]),
                gd2_ref[...], ed2_ref[...])
  d = _bn_lanes(jnp.tanh(dot(wd3_ref[...], d) + bd3_ref[...]),
                gd3_ref[...], ed3_ref[...])
  f = jnp.concatenate([h3, d], axis=0)
  zf = _bn_lanes(jnp.tanh(dot(wf_ref[...], f) + bf_ref[...]),
                 gf_ref[...], ef_ref[...])
  oa = jnp.tanh(dot(wa_ref[...], zf) + ba_ref[...])
  o_ref[...] = wo_ref[0, 0] * oa + bo_ref[0, 0]


def _full_spec(shape):
  return pl.BlockSpec(shape, lambda: tuple(0 for _ in shape))


def kernel(x, params, conns):
  p = params

  xt = x.T
  genet = xt[:_NG]
  drugt = xt[_NG:_NG + _DD]

  sel1 = conns["cols1"][::_K]
  sel2 = conns["cols2"][::_K * _K] // _K

  idx1 = jnp.zeros((_T1P * _F1,), jnp.int32).at[:_T1 * _F1].set(sel1)
  par1 = jnp.zeros((_T1P, 128), jnp.float32)
  par1 = par1.at[:_T1, :_F1 * _K].set(p["w1"].reshape(_T1, _F1 * _K))
  par1 = par1.at[:_T1, 96:102].set(p["b1"].reshape(_T1, _K))
  par1 = par1.at[:_T1, 102:108].set(p["g1"].reshape(_T1, _K))
  par1 = par1.at[:_T1, 108:114].set(p["be1"].reshape(_T1, _K))
  h1 = _sc_layer(genet, idx1.reshape(-1, 2 * _F1), par1.reshape(-1),
                 n_in=_F1, terms=_T1P, tpc=2, par_w=128)

  rid2 = (sel2.reshape(_T2, _F2)[:, :, None] * _K
          + jnp.arange(_K, dtype=jnp.int32)[None, None, :]).reshape(_T2, _F2 * _K)
  idx2 = jnp.zeros((_T2P, _F2 * _K), jnp.int32).at[:_T2].set(rid2)
  w2r = p["w2"].reshape(_T2, _F2, _K, _K).transpose(0, 1, 3, 2).reshape(_T2, 288)
  par2 = jnp.zeros((_T2P, 320), jnp.float32)
  par2 = par2.at[:_T2, :288].set(w2r)
  par2 = par2.at[:_T2, 288:294].set(p["b2"].reshape(_T2, _K))
  par2 = par2.at[:_T2, 294:300].set(p["g2"].reshape(_T2, _K))
  par2 = par2.at[:_T2, 300:306].set(p["be2"].reshape(_T2, _K))
  h2 = _sc_layer(h1, idx2, par2.reshape(-1), n_in=_F2 * _K, terms=_T2P,
                 tpc=1, par_w=320)

  w3t = jnp.zeros((_K, _T2P * _K), jnp.float32).at[:, :_T2 * _K].set(
      p["w3"].reshape(_T2, _K, _K).transpose(1, 0, 2).reshape(_K, _T2 * _K))
  args = [
      h2, drugt,
      w3t, p["b3"][:, None], p["g3"][:, None], p["be3"][:, None],
      p["Wd1"].T, p["bd1"][:, None], p["gd1"][:, None], p["bed1"][:, None],
      p["Wd2"].T, p["bd2"][:, None], p["gd2"][:, None], p["bed2"][:, None],
      p["Wd3"].T, p["bd3"][:, None], p["gd3"][:, None], p["bed3"][:, None],
      p["Wf"].T, p["bf"][:, None], p["gf"][:, None], p["bef"][:, None],
      p["Wa"].T, p["ba"][:, None], p["Wo"], p["bo"][:, None],
  ]
  out = pl.pallas_call(
      _head_body,
      in_specs=[_full_spec(tuple(a.shape)) for a in args],
      out_specs=_full_spec((1, _B)),
      out_shape=jax.ShapeDtypeStruct((1, _B), jnp.float32),
  )(*args)
  return out.reshape(_B, 1)


# speedup vs baseline: 1.6792x; 1.6792x over previous
import functools

import jax
import jax.numpy as jnp
from jax import lax
from jax.experimental import pallas as pl
from jax.experimental.pallas import tpu as pltpu
from jax.experimental.pallas import tpu_sc as plsc

_B = 512
_NG = 3008
_DD = 2048
_K = 6
_T1, _T2 = 1000, 250
_F1, _F2 = 16, 8
_T1P = 1024
_T2P = 256
_NW = 32


def _sc_gather(table, idx2, n_pad, d, chunk, nchunks):
  per_w = nchunks * chunk
  mesh = plsc.VectorSubcoreMesh(core_axis_name="c", subcore_axis_name="s")

  @functools.partial(
      pl.kernel,
      out_type=jax.ShapeDtypeStruct((n_pad, d), jnp.float32),
      mesh=mesh,
      scratch_types=[
          pltpu.VMEM((nchunks, chunk), jnp.int32),
          pltpu.VMEM((chunk, d), jnp.float32),
          pltpu.VMEM((chunk, d), jnp.float32),
          pltpu.SemaphoreType.DMA,
          pltpu.SemaphoreType.DMA,
      ],
  )
  def gather_k(table_hbm, idx_hbm, out_hbm, idx_v, buf0, buf1, sem0, sem1):
    wid = lax.axis_index("s") * 2 + lax.axis_index("c")
    rowbase = wid * per_w
    pltpu.sync_copy(idx_hbm.at[pl.ds(wid * nchunks, nchunks)], idx_v)
    bufs = (buf0, buf1)
    sems = (sem0, sem1)
    cps = [None, None]
    cps[0] = pltpu.async_copy(table_hbm.at[idx_v.at[0]], buf0, sem0)
    for c in range(nchunks):
      cur = c % 2
      nxt = (c + 1) % 2
      if c + 1 < nchunks:
        cps[nxt] = pltpu.async_copy(
            table_hbm.at[idx_v.at[c + 1]], bufs[nxt], sems[nxt])
      cps[cur].wait()
      pltpu.sync_copy(bufs[cur], out_hbm.at[pl.ds(rowbase + c * chunk, chunk)])

  return gather_k(table, idx2)


def _bn_lanes(z, ga, be):
  m = jnp.mean(z, axis=-1, keepdims=True)
  dd = z - m
  v = jnp.mean(dd * dd, axis=-1, keepdims=True)
  return ga * dd * lax.rsqrt(v + 1e-5) + be


def _l1_body(g_ref, w_ref, b_ref, ga_ref, be_ref, o_ref):
  tb = w_ref.shape[0]
  g = g_ref[...].reshape(tb, _F1, _B)
  w = w_ref[...].reshape(tb, _F1, _K)
  acc = b_ref[...][:, :, None] * jnp.ones((tb, _K, _B), jnp.float32)
  for i in range(_F1):
    acc = acc + w[:, i, :, None] * g[:, i, None, :]
  z = jnp.tanh(acc)
  h = _bn_lanes(z, ga_ref[...][:, :, None], be_ref[...][:, :, None])
  o_ref[...] = h.reshape(tb * _K, _B)


def _l2_body(g_ref, w_ref, b_ref, ga_ref, be_ref, o_ref):
  ob = w_ref.shape[0]
  g = g_ref[...].reshape(ob, _F2, _K * _B)
  w = w_ref[...].reshape(ob, _F2, _K, _K)
  acc = b_ref[...][:, :, None] * jnp.ones((ob, _K, _B), jnp.float32)
  for j in range(_F2):
    for q in range(_K):
      acc = acc + (w[:, j, :, q][:, :, None]
                   * g[:, j, q * _B:(q + 1) * _B][:, None, :])
  z = jnp.tanh(acc)
  h = _bn_lanes(z, ga_ref[...][:, :, None], be_ref[...][:, :, None])
  o_ref[...] = h.reshape(ob * _K, _B)


def _dmlp_body(dr_ref,
               wd1_ref, bd1_ref, gd1_ref, ed1_ref,
               wd2_ref, bd2_ref, gd2_ref, ed2_ref,
               wd3_ref, bd3_ref, gd3_ref, ed3_ref, o_ref):
  def dot(a, b):
    return lax.dot(a, b, precision=lax.Precision.HIGHEST,
                   preferred_element_type=jnp.float32)

  d = dr_ref[...]
  d = _bn_lanes(jnp.tanh(dot(wd1_ref[...], d) + bd1_ref[...]),
                gd1_ref[...], ed1_ref[...])
  d = _bn_lanes(jnp.tanh(dot(wd2_ref[...], d) + bd2_ref[...]),
                gd2_ref[...], ed2_ref[...])
  d = _bn_lanes(jnp.tanh(dot(wd3_ref[...], d) + bd3_ref[...]),
                gd3_ref[...], ed3_ref[...])
  o_ref[...] = d


def _head_body(h2a_ref, h2b_ref, d_ref, w3a_ref, w3b_ref,
               b3_ref, g3_ref, e3_ref,
               wf_ref, bf_ref, gf_ref, ef_ref,
               wa_ref, ba_ref, wo_ref, bo_ref, o_ref):
  def dot(a, b):
    return lax.dot(a, b, precision=lax.Precision.HIGHEST,
                   preferred_element_type=jnp.float32)

  z3 = dot(w3a_ref[...], h2a_ref[...]) + dot(w3b_ref[...], h2b_ref[...])
  h3 = _bn_lanes(jnp.tanh(z3 + b3_ref[...]), g3_ref[...], e3_ref[...])
  f = jnp.concatenate([h3, d_ref[...]], axis=0)
  zf = _bn_lanes(jnp.tanh(dot(wf_ref[...], f) + bf_ref[...]),
                 gf_ref[...], ef_ref[...])
  oa = jnp.tanh(dot(wa_ref[...], zf) + ba_ref[...])
  o_ref[...] = wo_ref[0, 0] * oa + bo_ref[0, 0]


def _full_spec(shape):
  return pl.BlockSpec(shape, lambda: tuple(0 for _ in shape))


def _contract(body, g, w, b, ga, be, nb, tb, gw, d):
  return pl.pallas_call(
      body,
      grid=(nb,),
      in_specs=[
          pl.BlockSpec((tb * gw, d), lambda i: (i, 0)),
          pl.BlockSpec((tb, w.shape[1]), lambda i: (i, 0)),
          pl.BlockSpec((tb, _K), lambda i: (i, 0)),
          pl.BlockSpec((tb, _K), lambda i: (i, 0)),
          pl.BlockSpec((tb, _K), lambda i: (i, 0)),
      ],
      out_specs=pl.BlockSpec((tb * _K, _B), lambda i: (i, 0)),
      out_shape=jax.ShapeDtypeStruct((nb * tb * _K, _B), jnp.float32),
  )(g, w, b, ga, be)


def kernel(x, params, conns):
  p = params

  xt = x.T
  genet = xt[:_NG]
  drugt = xt[_NG:_NG + _DD]

  sel1 = conns["cols1"][::_K]
  sel2 = conns["cols2"][::_K * _K] // _K

  dargs = [
      drugt,
      p["Wd1"].T, p["bd1"][:, None], p["gd1"][:, None], p["bed1"][:, None],
      p["Wd2"].T, p["bd2"][:, None], p["gd2"][:, None], p["bed2"][:, None],
      p["Wd3"].T, p["bd3"][:, None], p["gd3"][:, None], p["bed3"][:, None],
  ]
  dml = pl.pallas_call(
      _dmlp_body,
      in_specs=[_full_spec(tuple(a.shape)) for a in dargs],
      out_specs=_full_spec((_K, _B)),
      out_shape=jax.ShapeDtypeStruct((_K, _B), jnp.float32),
  )(*dargs)

  idx1 = jnp.zeros((_T1P * _F1,), jnp.int32).at[:_T1 * _F1].set(sel1)
  idx1c = idx1.reshape(-1, 64)
  nr1 = _T1P * _F1 // 2
  g1a = _sc_gather(genet, idx1c[:nr1 // 64], nr1, _B, 64, 4)
  g1b = _sc_gather(genet, idx1c[nr1 // 64:], nr1, _B, 64, 4)

  w1m = jnp.zeros((_T1P, _F1 * _K), jnp.float32).at[:_T1].set(
      p["w1"].reshape(_T1, _F1 * _K))
  b1m = jnp.zeros((_T1P, _K), jnp.float32).at[:_T1].set(p["b1"].reshape(_T1, _K))
  ga1m = jnp.zeros((_T1P, _K), jnp.float32).at[:_T1].set(p["g1"].reshape(_T1, _K))
  be1m = jnp.zeros((_T1P, _K), jnp.float32).at[:_T1].set(p["be1"].reshape(_T1, _K))

  tb1 = 32
  hp = _T1P // 2
  h1a = _contract(_l1_body, g1a, w1m[:hp], b1m[:hp], ga1m[:hp], be1m[:hp],
                  hp // tb1, tb1, _F1, _B)
  h1b = _contract(_l1_body, g1b, w1m[hp:], b1m[hp:], ga1m[hp:], be1m[hp:],
                  hp // tb1, tb1, _F1, _B)
  h1tab = jnp.concatenate([h1a, h1b], axis=0).reshape(_T1P, _K * _B)

  idx2 = jnp.zeros((_T2P * _F2,), jnp.int32).at[:_T2 * _F2].set(sel2)
  idx2c = idx2.reshape(-1, 16)
  nr2 = _T2P * _F2 // 2
  g2a = _sc_gather(h1tab, idx2c[:nr2 // 16], nr2, _K * _B, 16, 2)
  g2b = _sc_gather(h1tab, idx2c[nr2 // 16:], nr2, _K * _B, 16, 2)

  w2m = jnp.zeros((_T2P, _F2 * _K * _K), jnp.float32).at[:_T2].set(
      p["w2"].reshape(_T2, _F2 * _K * _K))
  b2m = jnp.zeros((_T2P, _K), jnp.float32).at[:_T2].set(p["b2"].reshape(_T2, _K))
  ga2m = jnp.zeros((_T2P, _K), jnp.float32).at[:_T2].set(p["g2"].reshape(_T2, _K))
  be2m = jnp.zeros((_T2P, _K), jnp.float32).at[:_T2].set(p["be2"].reshape(_T2, _K))

  ob2 = 32
  qp = _T2P // 2
  h2a = _contract(_l2_body, g2a, w2m[:qp], b2m[:qp], ga2m[:qp], be2m[:qp],
                  qp // ob2, ob2, _F2, _K * _B)
  h2b = _contract(_l2_body, g2b, w2m[qp:], b2m[qp:], ga2m[qp:], be2m[qp:],
                  qp // ob2, ob2, _F2, _K * _B)

  w3t = jnp.zeros((_K, _T2P * _K), jnp.float32).at[:, :_T2 * _K].set(
      p["w3"].reshape(_T2, _K, _K).transpose(1, 0, 2).reshape(_K, _T2 * _K))
  hw = _T2P * _K // 2
  args = [
      h2a, h2b, dml, w3t[:, :hw], w3t[:, hw:],
      p["b3"][:, None], p["g3"][:, None], p["be3"][:, None],
      p["Wf"].T, p["bf"][:, None], p["gf"][:, None], p["bef"][:, None],
      p["Wa"].T, p["ba"][:, None], p["Wo"], p["bo"][:, None],
  ]
  out = pl.pallas_call(
      _head_body,
      in_specs=[_full_spec(tuple(a.shape)) for a in args],
      out_specs=_full_spec((1, _B)),
      out_shape=jax.ShapeDtypeStruct((1, _B), jnp.float32),
  )(*args)
  return out.reshape(_B, 1)
